# baseline (speedup 1.0000x reference)
import functools

import jax
import jax.numpy as jnp
from jax import lax
from jax.experimental import pallas as pl
from jax.experimental.pallas import tpu as pltpu
from jax.experimental.pallas import tpu_sc as plsc

LANES = 16


def _xlane_rotate(v, perm):
    dnums = lax.GatherDimensionNumbers(
        offset_dims=(), collapsed_slice_dims=(0,), start_index_map=(0,))
    return lax.gather(v, perm[:, None], dnums, (1,),
                      mode=lax.GatherScatterMode.PROMISE_IN_BOUNDS)


@functools.partial(jax.jit, static_argnames=("rows", "cols", "k_out", "chunk"))
def _sc_pool(x2, idxf, maskf, *, rows, cols, k_out, chunk):
    info = plsc.get_sparse_core_info()
    num_workers = info.num_cores * info.num_subcores
    rows_per_w = rows // num_workers
    nchunks = rows_per_w // chunk
    nvec = cols // LANES

    mesh = plsc.VectorSubcoreMesh(core_axis_name="c", subcore_axis_name="s")

    @functools.partial(
        pl.kernel,
        out_type=jax.ShapeDtypeStruct((rows, k_out), jnp.float32),
        mesh=mesh,
        compiler_params=pltpu.CompilerParams(use_tc_tiling_on_sc=False,
                                             needs_layout_passes=False),
        scratch_types=[
            pltpu.VMEM((cols,), jnp.int32),
            pltpu.VMEM((cols,), jnp.float32),
            pltpu.VMEM((chunk, cols), jnp.float32),
            pltpu.VMEM((chunk, k_out), jnp.float32),
        ],
    )
    def k(x_hbm, idx_hbm, mask_hbm, out_hbm, idx_v, mask_v, xbuf, obuf):
        wid = lax.axis_index("s") * info.num_cores + lax.axis_index("c")
        base = wid * rows_per_w
        pltpu.sync_copy(idx_hbm, idx_v)
        pltpu.sync_copy(mask_hbm, mask_v)
        idx_regs = [idx_v[pl.ds(LANES * j, LANES)] for j in range(nvec)]
        mask_regs = [mask_v[pl.ds(LANES * j, LANES)] for j in range(nvec)]
        lanes = lax.iota(jnp.int32, LANES)
        perm8 = lax.rem(lanes + 8, jnp.full((LANES,), 16, jnp.int32))
        perm4 = lax.rem(lanes + 4, jnp.full((LANES,), 16, jnp.int32))
        out_col = lax.rem(lanes, jnp.full((LANES,), k_out, jnp.int32))
        store_mask = lanes < k_out

        def chunk_body(c, carry):
            pltpu.sync_copy(x_hbm.at[pl.ds(base + c * chunk, chunk)], xbuf)

            def row_body(r, carry2):
                rsplat = jnp.full((LANES,), r, jnp.int32)
                vals = [
                    plsc.load_gather(xbuf, [rsplat, idx_regs[j]]) * mask_regs[j]
                    for j in range(nvec)
                ]
                while len(vals) > 1:
                    vals = [jnp.maximum(vals[2 * i], vals[2 * i + 1])
                            for i in range(len(vals) // 2)]
                acc = vals[0]
                acc = jnp.maximum(acc, _xlane_rotate(acc, perm8))
                acc = jnp.maximum(acc, _xlane_rotate(acc, perm4))
                plsc.store_scatter(obuf, [rsplat, out_col], acc,
                                   mask=store_mask)
                return carry2

            lax.fori_loop(0, chunk, row_body, 0, unroll=False)
            pltpu.sync_copy(obuf, out_hbm.at[pl.ds(base + c * chunk, chunk)])
            return carry

        lax.fori_loop(0, nchunks, chunk_body, 0, unroll=False)

    return k(x2, idxf, maskf)


def kernel(input_images, indices, mask):
    b, f, c = input_images.shape
    l, k_out = indices.shape
    rows = b * f
    x2 = input_images.reshape(rows, c)
    idxf = indices.reshape(l * k_out).astype(jnp.int32)
    maskf = mask.reshape(l * k_out).astype(jnp.float32)
    out2 = _sc_pool(x2, idxf, maskf, rows=rows, cols=l * k_out,
                    k_out=k_out, chunk=128)
    return out2.reshape(b, f, k_out)


# speedup vs baseline: 1.0355x; 1.0355x over previous
import functools

import jax
import jax.numpy as jnp
from jax import lax
from jax.experimental import pallas as pl
from jax.experimental.pallas import tpu as pltpu
from jax.experimental.pallas import tpu_sc as plsc

LANES = 16


def _xlane_rotate(v, perm):
    dnums = lax.GatherDimensionNumbers(
        offset_dims=(), collapsed_slice_dims=(0,), start_index_map=(0,))
    return lax.gather(v, perm[:, None], dnums, (1,),
                      mode=lax.GatherScatterMode.PROMISE_IN_BOUNDS)


@functools.partial(jax.jit, static_argnames=("rows", "cols", "k_out", "chunk"))
def _sc_pool(x2, idxf, maskf, *, rows, cols, k_out, chunk):
    info = plsc.get_sparse_core_info()
    num_workers = info.num_cores * info.num_subcores
    rows_per_w = rows // num_workers
    nchunks = rows_per_w // chunk
    nvec = cols // LANES

    mesh = plsc.VectorSubcoreMesh(core_axis_name="c", subcore_axis_name="s")

    @functools.partial(
        pl.kernel,
        out_type=jax.ShapeDtypeStruct((rows, k_out), jnp.float32),
        mesh=mesh,
        compiler_params=pltpu.CompilerParams(use_tc_tiling_on_sc=False,
                                             needs_layout_passes=False),
        scratch_types=[
            pltpu.VMEM((cols,), jnp.int32),
            pltpu.VMEM((cols,), jnp.float32),
            pltpu.VMEM((chunk, cols), jnp.float32),
            pltpu.VMEM((chunk, cols), jnp.float32),
            pltpu.VMEM((rows_per_w, k_out), jnp.float32),
            pltpu.SemaphoreType.DMA,
            pltpu.SemaphoreType.DMA,
        ],
    )
    def k(x_hbm, idx_hbm, mask_hbm, out_hbm,
          idx_v, mask_v, xbuf0, xbuf1, oslab, sem0, sem1):
        wid = lax.axis_index("s") * info.num_cores + lax.axis_index("c")
        base = wid * rows_per_w
        pltpu.sync_copy(idx_hbm, idx_v)
        pltpu.sync_copy(mask_hbm, mask_v)
        idx_regs = [idx_v[pl.ds(LANES * j, LANES)] for j in range(nvec)]
        mask_regs = [mask_v[pl.ds(LANES * j, LANES)] for j in range(nvec)]
        lanes = lax.iota(jnp.int32, LANES)
        perm8 = lax.rem(lanes + 8, jnp.full((LANES,), 16, jnp.int32))
        perm4 = lax.rem(lanes + 4, jnp.full((LANES,), 16, jnp.int32))
        out_col = lax.rem(lanes, jnp.full((LANES,), k_out, jnp.int32))
        store_mask = lanes < k_out

        xbufs = (xbuf0, xbuf1)
        sems = (sem0, sem1)
        pltpu.async_copy(x_hbm.at[pl.ds(base, chunk)], xbuf0, sem0)
        pltpu.async_copy(x_hbm.at[pl.ds(base + chunk, chunk)], xbuf1, sem1)

        @pl.loop(0, nchunks, step=2)
        def outer(c):
            for b in range(2):
                cur = c + b
                pltpu.make_async_copy(
                    x_hbm.at[pl.ds(base, chunk)], xbufs[b], sems[b]).wait()
                obase = cur * chunk

                @plsc.parallel_loop(0, chunk, unroll=4)
                def row_body(r):
                    rsplat = jnp.full((LANES,), r, jnp.int32)
                    vals = [
                        plsc.load_gather(xbufs[b], [rsplat, idx_regs[j]])
                        * mask_regs[j]
                        for j in range(nvec)
                    ]
                    while len(vals) > 1:
                        vals = [jnp.maximum(vals[2 * i], vals[2 * i + 1])
                                for i in range(len(vals) // 2)]
                    acc = vals[0]
                    acc = jnp.maximum(acc, _xlane_rotate(acc, perm8))
                    acc = jnp.maximum(acc, _xlane_rotate(acc, perm4))
                    orow = jnp.full((LANES,), obase + r, jnp.int32)
                    plsc.store_scatter(oslab, [orow, out_col], acc,
                                       mask=store_mask)

                @pl.when(cur + 2 < nchunks)
                def _():
                    pltpu.async_copy(
                        x_hbm.at[pl.ds(base + (cur + 2) * chunk, chunk)],
                        xbufs[b], sems[b])

        pltpu.sync_copy(oslab, out_hbm.at[pl.ds(base, rows_per_w)])

    return k(x2, idxf, maskf)


def kernel(input_images, indices, mask):
    b, f, c = input_images.shape
    l, k_out = indices.shape
    rows = b * f
    x2 = input_images.reshape(rows, c)
    idxf = indices.reshape(l * k_out).astype(jnp.int32)
    maskf = mask.reshape(l * k_out).astype(jnp.float32)
    out2 = _sc_pool(x2, idxf, maskf, rows=rows, cols=l * k_out,
                    k_out=k_out, chunk=128)
    return out2.reshape(b, f, k_out)


# speedup vs baseline: 1.1642x; 1.1243x over previous
import functools

import jax
import jax.numpy as jnp
from jax import lax
from jax.experimental import pallas as pl
from jax.experimental.pallas import tpu as pltpu
from jax.experimental.pallas import tpu_sc as plsc

LANES = 16


def _xlane_rotate(v, perm):
    dnums = lax.GatherDimensionNumbers(
        offset_dims=(), collapsed_slice_dims=(0,), start_index_map=(0,))
    return lax.gather(v, perm[:, None], dnums, (1,),
                      mode=lax.GatherScatterMode.PROMISE_IN_BOUNDS)


@functools.partial(jax.jit, static_argnames=("rows", "cols", "k_out", "chunk"))
def _sc_pool(x2, idxf, maskf, *, rows, cols, k_out, chunk):
    info = plsc.get_sparse_core_info()
    num_workers = info.num_cores * info.num_subcores
    rows_per_w = rows // num_workers
    nchunks = rows_per_w // chunk
    nvec = cols // LANES
    oflat_per_w = rows_per_w * k_out // 128

    mesh = plsc.VectorSubcoreMesh(core_axis_name="c", subcore_axis_name="s")

    @functools.partial(
        pl.kernel,
        out_type=jax.ShapeDtypeStruct((rows * k_out // 128, 128), jnp.float32),
        mesh=mesh,
        compiler_params=pltpu.CompilerParams(use_tc_tiling_on_sc=True,
                                             needs_layout_passes=False),
        scratch_types=[
            pltpu.VMEM((cols,), jnp.int32),
            pltpu.VMEM((cols,), jnp.float32),
            pltpu.VMEM((chunk, cols), jnp.float32),
            pltpu.VMEM((chunk, cols), jnp.float32),
            pltpu.VMEM((oflat_per_w, 128), jnp.float32),
            pltpu.SemaphoreType.DMA,
            pltpu.SemaphoreType.DMA,
        ],
    )
    def k(x_hbm, idx_hbm, mask_hbm, out_hbm,
          idx_v, mask_v, xbuf0, xbuf1, oslab, sem0, sem1):
        wid = lax.axis_index("s") * info.num_cores + lax.axis_index("c")
        base = wid * rows_per_w
        pltpu.sync_copy(idx_hbm, idx_v)
        pltpu.sync_copy(mask_hbm, mask_v)
        idx_regs = [idx_v[pl.ds(LANES * j, LANES)] for j in range(nvec)]
        mask_regs = [mask_v[pl.ds(LANES * j, LANES)] for j in range(nvec)]
        lanes = lax.iota(jnp.int32, LANES)
        perm8 = lax.rem(lanes + 8, jnp.full((LANES,), 16, jnp.int32))
        perm4 = lax.rem(lanes + 4, jnp.full((LANES,), 16, jnp.int32))
        store_mask = lanes < k_out

        xbufs = (xbuf0, xbuf1)
        sems = (sem0, sem1)
        pltpu.async_copy(x_hbm.at[pl.ds(base, chunk)], xbuf0, sem0)
        pltpu.async_copy(x_hbm.at[pl.ds(base + chunk, chunk)], xbuf1, sem1)

        @pl.loop(0, nchunks, step=2)
        def outer(c):
            for b in range(2):
                cur = c + b
                pltpu.make_async_copy(
                    x_hbm.at[pl.ds(base, chunk)], xbufs[b], sems[b]).wait()
                obase = cur * chunk

                @plsc.parallel_loop(0, chunk, unroll=4)
                def row_body(r):
                    rsplat = jnp.full((LANES,), r, jnp.int32)
                    vals = [
                        plsc.load_gather(xbufs[b], [rsplat, idx_regs[j]])
                        * mask_regs[j]
                        for j in range(nvec)
                    ]
                    while len(vals) > 1:
                        vals = [jnp.maximum(vals[2 * i], vals[2 * i + 1])
                                for i in range(len(vals) // 2)]
                    acc = vals[0]
                    acc = jnp.maximum(acc, _xlane_rotate(acc, perm8))
                    acc = jnp.maximum(acc, _xlane_rotate(acc, perm4))
                    pos = jnp.full((LANES,), (obase + r) * k_out,
                                   jnp.int32) + lanes
                    plsc.store_scatter(
                        oslab,
                        [lax.shift_right_logical(pos, 7),
                         lax.bitwise_and(pos, jnp.full((LANES,), 127,
                                                       jnp.int32))],
                        acc, mask=store_mask)

                @pl.when(cur + 2 < nchunks)
                def _():
                    pltpu.async_copy(
                        x_hbm.at[pl.ds(base + (cur + 2) * chunk, chunk)],
                        xbufs[b], sems[b])

        pltpu.sync_copy(oslab, out_hbm.at[pl.ds(wid * oflat_per_w,
                                                oflat_per_w)])

    return k(x2, idxf, maskf)


def kernel(input_images, indices, mask):
    b, f, c = input_images.shape
    l, k_out = indices.shape
    rows = b * f
    x2 = input_images.reshape(rows, c)
    idxf = indices.reshape(l * k_out).astype(jnp.int32)
    maskf = mask.reshape(l * k_out).astype(jnp.float32)
    out2 = _sc_pool(x2, idxf, maskf, rows=rows, cols=l * k_out,
                    k_out=k_out, chunk=128)
    return out2.reshape(rows, k_out).reshape(b, f, k_out)


# speedup vs baseline: 2.4203x; 2.0789x over previous
import functools

import jax
import jax.numpy as jnp
from jax import lax
from jax.experimental import pallas as pl
from jax.experimental.pallas import tpu as pltpu
from jax.experimental.pallas import tpu_sc as plsc

LANES = 16


def _xlane_rotate(v, perm):
    dnums = lax.GatherDimensionNumbers(
        offset_dims=(), collapsed_slice_dims=(0,), start_index_map=(0,))
    return lax.gather(v, perm[:, None], dnums, (1,),
                      mode=lax.GatherScatterMode.PROMISE_IN_BOUNDS)


@functools.partial(jax.jit, static_argnames=("rows", "cols", "k_out", "chunk"))
def _sc_pool(x2, idxf, maskf, *, rows, cols, k_out, chunk):
    info = plsc.get_sparse_core_info()
    num_workers = info.num_cores * info.num_subcores
    rows_per_w = rows // num_workers
    nchunks = rows_per_w // chunk
    nvec = cols // LANES
    oflat_per_w = rows_per_w * k_out // 128

    mesh = plsc.VectorSubcoreMesh(core_axis_name="c", subcore_axis_name="s")

    @functools.partial(
        pl.kernel,
        out_type=jax.ShapeDtypeStruct((rows * k_out // 128, 128), jnp.float32),
        mesh=mesh,
        compiler_params=pltpu.CompilerParams(use_tc_tiling_on_sc=True,
                                             needs_layout_passes=False),
        scratch_types=[
            pltpu.VMEM((cols,), jnp.int32),
            pltpu.VMEM((cols,), jnp.float32),
            pltpu.VMEM((chunk, cols), jnp.float32),
            pltpu.VMEM((chunk, cols), jnp.float32),
            pltpu.VMEM((oflat_per_w, 128), jnp.float32),
            pltpu.SemaphoreType.DMA,
            pltpu.SemaphoreType.DMA,
        ],
    )
    def k(x_hbm, idx_hbm, mask_hbm, out_hbm,
          idx_v, mask_v, xbuf0, xbuf1, oslab, sem0, sem1):
        wid = lax.axis_index("s") * info.num_cores + lax.axis_index("c")
        base = wid * rows_per_w
        pltpu.sync_copy(idx_hbm, idx_v)
        pltpu.sync_copy(mask_hbm, mask_v)
        mask_regs = [mask_v[pl.ds(LANES * j, LANES)] for j in range(nvec)]
        idx_regs = [idx_v[pl.ds(LANES * j, LANES)] for j in range(nvec)]
        lanes = lax.iota(jnp.int32, LANES)
        perm8 = lax.rem(lanes + 8, jnp.full((LANES,), 16, jnp.int32))
        perm4 = lax.rem(lanes + 4, jnp.full((LANES,), 16, jnp.int32))
        store_mask = lanes < k_out

        xbufs = (xbuf0, xbuf1)
        sems = (sem0, sem1)
        pltpu.async_copy(x_hbm.at[pl.ds(base, chunk)], xbuf0, sem0)
        pltpu.async_copy(x_hbm.at[pl.ds(base + chunk, chunk)], xbuf1, sem1)

        @pl.loop(0, nchunks, step=2)
        def outer(c):
            for b in range(2):
                cur = c + b
                pltpu.make_async_copy(
                    x_hbm.at[pl.ds(base, chunk)],
                    xbufs[b], sems[b]).wait()
                obase = cur * chunk

                @plsc.parallel_loop(0, chunk, unroll=4)
                def row_body(r):
                    rsplat = jnp.full((LANES,), r, jnp.int32)
                    vals = [
                        plsc.load_gather(xbufs[b], [rsplat, idx_regs[j]])
                        * mask_regs[j]
                        for j in range(nvec)
                    ]
                    while len(vals) > 1:
                        vals = [jnp.maximum(vals[2 * i], vals[2 * i + 1])
                                for i in range(len(vals) // 2)]
                    acc = vals[0]
                    acc = jnp.maximum(acc, _xlane_rotate(acc, perm8))
                    acc = jnp.maximum(acc, _xlane_rotate(acc, perm4))
                    f_loc = obase + r
                    srow = jnp.full((LANES,), (f_loc // 128) * k_out,
                                    jnp.int32) + lanes
                    scol = jnp.full((LANES,), f_loc % 128, jnp.int32)
                    plsc.store_scatter(oslab, [srow, scol], acc,
                                       mask=store_mask)

                @pl.when(cur + 2 < nchunks)
                def _():
                    pltpu.async_copy(
                        x_hbm.at[pl.ds(base + (cur + 2) * chunk, chunk)],
                        xbufs[b], sems[b])

        pltpu.sync_copy(oslab, out_hbm.at[pl.ds(wid * oflat_per_w,
                                                oflat_per_w)])

    return k(x2, idxf, maskf)


def kernel(input_images, indices, mask):
    b, f, c = input_images.shape
    l, k_out = indices.shape
    rows = b * f
    x2 = input_images.reshape(rows, c)
    idxf = indices.reshape(l * k_out).astype(jnp.int32)
    maskf = mask.reshape(l * k_out).astype(jnp.float32)
    out2 = _sc_pool(x2, idxf, maskf, rows=rows, cols=l * k_out,
                    k_out=k_out, chunk=128)
    out4 = out2.reshape(b, f // 128, k_out, 128)
    return out4.transpose(0, 1, 3, 2).reshape(b, f, k_out)
